# baseline (speedup 1.0000x reference)
import jax
import jax.numpy as jnp
from jax import lax
from jax.experimental import pallas as pl
from jax.experimental.pallas import tpu as pltpu

_THR = 0.01
_NP_RATIO = 3.0
_HE_RATIO = 0.005

_SUB = 1152
_LANE = 128
_HW = _SUB * _LANE
_ROWS = 24
_G = 8
_STEPS = _ROWS // _G
_CH = 16
_NCH = _SUB // _CH


def _count_ge(ref, pivot16):
    acc = jnp.zeros((_G, _CH, _LANE), jnp.int16)
    for j in range(_NCH):
        chunk = ref[:, j * _CH : (j + 1) * _CH, :]
        acc = acc + (chunk >= pivot16[:, None, None]).astype(jnp.int16)
    return jnp.sum(acc.astype(jnp.int32), axis=(1, 2))


def _ohem_body(x_ref, y_ref, out_ref, acc_ref, hi_ref, lo_ref, z_ref):
    r = pl.program_id(0)
    d = x_ref[0] - y_ref[0]
    loss = d * d
    neg = loss >= _THR
    bits = jnp.where(neg, lax.bitcast_convert_type(loss, jnp.int32), 0)
    hi_ref[...] = (bits >> 16).astype(jnp.int16)
    lo_ref[...] = ((bits & 0xFFFF) - 32768).astype(jnp.int16)

    nneg = jnp.sum(neg.astype(jnp.int32), axis=(1, 2))
    npos = _HW - nneg
    nneg_f = nneg.astype(jnp.float32)
    npos_f = npos.astype(jnp.float32)
    nhe = jnp.where(nneg_f > _NP_RATIO * npos_f, _NP_RATIO * npos_f, nneg_f)
    nhe = jnp.maximum(nhe, jnp.float32(_HE_RATIO * float(_HW)))
    k = jnp.floor(nhe).astype(jnp.int32)
    k_eff = jnp.minimum(k, nneg)

    def step_a(_, carry):
        lo, hi = carry
        mid = lo + ((hi - lo) >> 1)
        cnt = _count_ge(hi_ref, mid.astype(jnp.int16))
        ge = cnt >= k_eff
        return jnp.where(ge, mid, lo), jnp.where(ge, hi, mid)

    p_star, _ = lax.fori_loop(
        0,
        15,
        step_a,
        (jnp.zeros((_G,), jnp.int32), jnp.full((_G,), 0x7F80, jnp.int32)),
        unroll=False,
    )

    p16 = p_star.astype(jnp.int16)[:, None, None]
    hi16 = hi_ref[...]
    z = jnp.where(
        hi16 > p16,
        jnp.int16(32767),
        jnp.where(hi16 < p16, jnp.int16(-32768), lo_ref[...]),
    )
    z_ref[...] = z

    def step_b(_, carry):
        lo, hi = carry
        mid = lo + ((hi - lo) >> 1)
        cnt = _count_ge(z_ref, mid.astype(jnp.int16))
        ge = cnt >= k_eff
        return jnp.where(ge, mid, lo), jnp.where(ge, hi, mid)

    m_star, _ = lax.fori_loop(
        0,
        16,
        step_b,
        (
            jnp.full((_G,), -32768, jnp.int32),
            jnp.full((_G,), 32768, jnp.int32),
        ),
        unroll=False,
    )

    t = (p_star << 16) | (m_star + 32768)
    bits32 = (hi_ref[...].astype(jnp.int32) << 16) | (
        lo_ref[...].astype(jnp.int32) + 32768
    )
    gt = bits32 > t[:, None, None]
    c_gt = jnp.sum(gt.astype(jnp.int32), axis=(1, 2))
    s_gt = jnp.sum(
        jnp.where(gt, lax.bitcast_convert_type(bits32, jnp.float32), 0.0),
        axis=(1, 2),
    )
    tval = lax.bitcast_convert_type(t, jnp.float32)
    s_top = s_gt + (k_eff - c_gt).astype(jnp.float32) * tval
    l_rows = jnp.where(nneg > 0, s_top / nhe, 0.0)
    l_sum = jnp.sum(l_rows)

    @pl.when(r == 0)
    def _():
        acc_ref[0, 0] = 0.0

    acc_ref[0, 0] += l_sum

    @pl.when(r == _STEPS - 1)
    def _():
        out_ref[0, 0] = acc_ref[0, 0] / jnp.float32(_ROWS)


def kernel(x, y):
    x2 = x.reshape(_STEPS, _G, _SUB, _LANE)
    y2 = y.reshape(_STEPS, _G, _SUB, _LANE)
    out = pl.pallas_call(
        _ohem_body,
        grid=(_STEPS,),
        in_specs=[
            pl.BlockSpec((1, _G, _SUB, _LANE), lambda r: (r, 0, 0, 0)),
            pl.BlockSpec((1, _G, _SUB, _LANE), lambda r: (r, 0, 0, 0)),
        ],
        out_specs=pl.BlockSpec(memory_space=pltpu.SMEM),
        out_shape=jax.ShapeDtypeStruct((1, 1), jnp.float32),
        scratch_shapes=[
            pltpu.SMEM((1, 1), jnp.float32),
            pltpu.VMEM((_G, _SUB, _LANE), jnp.int16),
            pltpu.VMEM((_G, _SUB, _LANE), jnp.int16),
            pltpu.VMEM((_G, _SUB, _LANE), jnp.int16),
        ],
        compiler_params=pltpu.CompilerParams(
            dimension_semantics=("arbitrary",),
        ),
    )(x2, y2)
    return out[0, 0]
